# baseline (speedup 1.0000x reference)
import functools

import jax
import jax.numpy as jnp
from jax import lax
from jax.experimental import pallas as pl
from jax.experimental.pallas import tpu as pltpu
from jax.experimental.pallas import tpu_sc as plsc

D = 32
BB = 128
L = 16


def _build(B, H):
    N = B * H
    info = plsc.get_sparse_core_info()
    NC, NS = info.num_cores, info.num_subcores
    NW = NC * NS
    n_units = N // BB
    u_per_w = n_units // NW
    assert n_units % NW == 0 and B % BB == 0
    idx_per_w = N // NW
    NBUF = 2
    n_steps = u_per_w // NBUF
    BT = B // BB

    mesh = plsc.VectorSubcoreMesh(core_axis_name="c", subcore_axis_name="s")

    @functools.partial(
        pl.kernel,
        mesh=mesh,
        out_type=jax.ShapeDtypeStruct((H, D // 8, BT, 8, BB), jnp.float32),
        compiler_params=pltpu.CompilerParams(
            use_tc_tiling_on_sc=False, needs_layout_passes=False
        ),
        scratch_types=[
            pltpu.VMEM((idx_per_w,), jnp.int32),
            pltpu.VMEM((NBUF, BB, D), jnp.float32),
            pltpu.VMEM((NBUF, D // 8, 8, BB), jnp.float32),
            pltpu.SemaphoreType.DMA,
            pltpu.SemaphoreType.DMA,
            pltpu.SemaphoreType.DMA,
            pltpu.SemaphoreType.DMA,
        ],
    )
    def k(idx_hbm, table_hbm, out_hbm, idx_v, rows_v, tr_v, g0, g1, o0, o1):
        gsem = (g0, g1)
        osem = (o0, o1)
        wid = lax.axis_index("s") * NC + lax.axis_index("c")
        base = wid * idx_per_w
        u0 = wid * u_per_w
        pltpu.sync_copy(idx_hbm.at[pl.ds(base, idx_per_w)], idx_v)

        lane = lax.iota(jnp.int32, L)
        rowsel = [lane + L * g for g in range(BB // L)]

        def gather_copy(ul, b):
            off = pl.multiple_of(ul * BB, BB)
            return pltpu.make_async_copy(
                table_hbm.at[idx_v.at[pl.ds(off, BB)]], rows_v.at[b], gsem[b]
            )

        def store_copy(ul, b):
            u = u0 + ul
            h = lax.div(u, BT)
            bt = lax.rem(u, BT)
            return pltpu.make_async_copy(
                tr_v.at[b], out_hbm.at[h, :, bt], osem[b]
            )

        def transpose(b):
            for c in range(D):
                csel = jnp.full((L,), c, jnp.int32)
                for g in range(BB // L):
                    v = plsc.load_gather(rows_v.at[b], [rowsel[g], csel])
                    tr_v[b, c // 8, c % 8, pl.ds(L * g, L)] = v

        for b in range(NBUF):
            gather_copy(b, b).start()

        for b in range(NBUF):
            gather_copy(b, b).wait()
            transpose(b)
            gather_copy(b + NBUF, b).start()
            store_copy(b, b).start()

        def body(s, carry):
            for b in range(NBUF):
                ul = s * NBUF + b
                gather_copy(ul, b).wait()
                store_copy(ul - NBUF, b).wait()
                transpose(b)
                gather_copy(ul + NBUF, b).start()
                store_copy(ul, b).start()
            return carry

        lax.fori_loop(1, n_steps - 1, body, 0)

        for b in range(NBUF):
            ul = (n_steps - 1) * NBUF + b
            gather_copy(ul, b).wait()
            store_copy(ul - NBUF, b).wait()
            transpose(b)
            store_copy(ul, b).start()
        for b in range(NBUF):
            ul = (n_steps - 1) * NBUF + b
            store_copy(ul, b).wait()

    return k


def kernel(token_ids, emb):
    B, H = token_ids.shape
    idx = token_ids.T.reshape(B * H).astype(jnp.int32)
    k = _build(B, H)
    out5 = k(idx, emb)
    return out5.transpose(2, 4, 0, 1, 3).reshape(B, H, D)


# speedup vs baseline: 1.1921x; 1.1921x over previous
import functools

import jax
import jax.numpy as jnp
from jax import lax
from jax.experimental import pallas as pl
from jax.experimental.pallas import tpu as pltpu
from jax.experimental.pallas import tpu_sc as plsc

D = 32
BB = 128
L = 16


def _build(B, H):
    N = B * H
    info = plsc.get_sparse_core_info()
    NC, NS = info.num_cores, info.num_subcores
    NW = NC * NS
    n_units = N // BB
    u_per_w = n_units // NW
    assert n_units % NW == 0 and B % BB == 0
    idx_per_w = N // NW
    NBUF = 2
    n_steps = u_per_w // NBUF
    BT = B // BB

    mesh = plsc.VectorSubcoreMesh(core_axis_name="c", subcore_axis_name="s")

    @functools.partial(
        pl.kernel,
        mesh=mesh,
        out_type=jax.ShapeDtypeStruct((H, D // 8, BT, 8, BB), jnp.float32),
        compiler_params=pltpu.CompilerParams(
            use_tc_tiling_on_sc=False, needs_layout_passes=False
        ),
        scratch_types=[
            pltpu.VMEM((idx_per_w,), jnp.int32),
            pltpu.VMEM((NBUF, BB, D), jnp.float32),
            pltpu.VMEM((NBUF, D // 8, 8, BB), jnp.float32),
            pltpu.SemaphoreType.DMA,
            pltpu.SemaphoreType.DMA,
            pltpu.SemaphoreType.DMA,
            pltpu.SemaphoreType.DMA,
        ],
    )
    def k(idx_hbm, table_hbm, out_hbm, idx_v, rows_v, tr_v, g0, g1, o0, o1):
        gsem = (g0, g1)
        osem = (o0, o1)
        wid = lax.axis_index("s") * NC + lax.axis_index("c")
        base = wid * idx_per_w
        u0 = wid * u_per_w
        pltpu.sync_copy(idx_hbm.at[pl.ds(base, idx_per_w)], idx_v)

        lane = lax.iota(jnp.int32, L)
        rowsel = [lane + L * g for g in range(BB // L)]

        def gather_copy(ul, b):
            off = pl.multiple_of(ul * BB, BB)
            return pltpu.make_async_copy(
                table_hbm.at[idx_v.at[pl.ds(off, BB)]], rows_v.at[b], gsem[b]
            )

        def store_copy(ul, b):
            u = u0 + ul
            h = lax.div(u, BT)
            bt = lax.rem(u, BT)
            return pltpu.make_async_copy(
                tr_v.at[b], out_hbm.at[h, :, bt], osem[b]
            )

        def transpose(b):
            for c in range(D):
                csel = jnp.full((L,), c, jnp.int32)
                vs = [
                    plsc.load_gather(rows_v.at[b], [rowsel[g], csel])
                    for g in range(BB // L)
                ]
                for g in range(BB // L):
                    tr_v[b, c // 8, c % 8, pl.ds(L * g, L)] = vs[g]

        for b in range(NBUF):
            gather_copy(b, b).start()

        for b in range(NBUF):
            gather_copy(b, b).wait()
            transpose(b)
            gather_copy(b + NBUF, b).start()
            store_copy(b, b).start()

        def body(s, carry):
            for b in range(NBUF):
                ul = s * NBUF + b
                gather_copy(ul, b).wait()
                store_copy(ul - NBUF, b).wait()
                transpose(b)
                gather_copy(ul + NBUF, b).start()
                store_copy(ul, b).start()
            return carry

        lax.fori_loop(1, n_steps - 1, body, 0)

        for b in range(NBUF):
            ul = (n_steps - 1) * NBUF + b
            gather_copy(ul, b).wait()
            store_copy(ul - NBUF, b).wait()
            transpose(b)
            store_copy(ul, b).start()
        for b in range(NBUF):
            ul = (n_steps - 1) * NBUF + b
            store_copy(ul, b).wait()

    return k


def kernel(token_ids, emb):
    B, H = token_ids.shape
    idx = token_ids.T.reshape(B * H).astype(jnp.int32)
    k = _build(B, H)
    out5 = k(idx, emb)
    return out5.transpose(2, 4, 0, 1, 3).reshape(B, H, D)


# speedup vs baseline: 1.2064x; 1.0120x over previous
import functools

import jax
import jax.numpy as jnp
from jax import lax
from jax.experimental import pallas as pl
from jax.experimental.pallas import tpu as pltpu
from jax.experimental.pallas import tpu_sc as plsc

D = 32
BB = 128
L = 16


def _build(B, H):
    N = B * H
    info = plsc.get_sparse_core_info()
    NC, NS = info.num_cores, info.num_subcores
    NW = NC * NS
    n_units = N // BB
    u_per_w = n_units // NW
    assert n_units % NW == 0 and B % BB == 0
    idx_per_w = N // NW
    NBUF = 2
    n_steps = u_per_w // NBUF
    BT = B // BB
    CT = D // 8
    TRW = D * BB

    mesh = plsc.VectorSubcoreMesh(core_axis_name="c", subcore_axis_name="s")

    @functools.partial(
        pl.kernel,
        mesh=mesh,
        out_type=jax.ShapeDtypeStruct((H, CT, BT, 8 * BB), jnp.float32),
        compiler_params=pltpu.CompilerParams(
            use_tc_tiling_on_sc=False, needs_layout_passes=False
        ),
        scratch_types=[
            pltpu.VMEM((idx_per_w,), jnp.int32),
            pltpu.VMEM((NBUF, BB, D), jnp.float32),
            pltpu.VMEM((NBUF, TRW), jnp.float32),
            pltpu.SemaphoreType.DMA,
            pltpu.SemaphoreType.DMA,
            pltpu.SemaphoreType.DMA,
            pltpu.SemaphoreType.DMA,
        ],
    )
    def k(idx_hbm, table_hbm, out_hbm, idx_v, rows_v, tr_v, g0, g1, o0, o1):
        gsem = (g0, g1)
        osem = (o0, o1)
        wid = lax.axis_index("s") * NC + lax.axis_index("c")
        base = wid * idx_per_w
        u0 = wid * u_per_w
        pltpu.sync_copy(idx_hbm.at[pl.ds(base, idx_per_w)], idx_v)

        lane = lax.iota(jnp.int32, L)
        cbase = [lane * BB, (lane + L) * BB]

        def gather_copy(ul, b):
            off = pl.multiple_of(ul * BB, BB)
            return pltpu.make_async_copy(
                table_hbm.at[idx_v.at[pl.ds(off, BB)]], rows_v.at[b], gsem[b]
            )

        def store_copies(ul, b):
            u = u0 + ul
            h = lax.div(u, BT)
            bt = lax.rem(u, BT)
            return [
                pltpu.make_async_copy(
                    tr_v.at[b, pl.ds(ct * 8 * BB, 8 * BB)],
                    out_hbm.at[h, ct, bt],
                    osem[b],
                )
                for ct in range(CT)
            ]

        def transpose(b):
            dst = tr_v.at[b]
            for j in range(BB):
                v0 = rows_v[b, j, pl.ds(0, L)]
                v1 = rows_v[b, j, pl.ds(L, L)]
                plsc.store_scatter(dst, [cbase[0] + j], v0)
                plsc.store_scatter(dst, [cbase[1] + j], v1)

        for b in range(NBUF):
            gather_copy(b, b).start()

        for b in range(NBUF):
            gather_copy(b, b).wait()
            transpose(b)
            gather_copy(b + NBUF, b).start()
            for c in store_copies(b, b):
                c.start()

        def body(s, carry):
            for b in range(NBUF):
                ul = s * NBUF + b
                gather_copy(ul, b).wait()
                for c in store_copies(ul - NBUF, b):
                    c.wait()
                transpose(b)
                gather_copy(ul + NBUF, b).start()
                for c in store_copies(ul, b):
                    c.start()
            return carry

        lax.fori_loop(1, n_steps - 1, body, 0)

        for b in range(NBUF):
            ul = (n_steps - 1) * NBUF + b
            gather_copy(ul, b).wait()
            for c in store_copies(ul - NBUF, b):
                c.wait()
            transpose(b)
            for c in store_copies(ul, b):
                c.start()
        for b in range(NBUF):
            ul = (n_steps - 1) * NBUF + b
            for c in store_copies(ul, b):
                c.wait()

    return k


def kernel(token_ids, emb):
    B, H = token_ids.shape
    idx = token_ids.T.reshape(B * H).astype(jnp.int32)
    k = _build(B, H)
    out4 = k(idx, emb)
    return (
        out4.reshape(H, D // 8, B // BB, 8, BB)
        .transpose(2, 4, 0, 1, 3)
        .reshape(B, H, D)
    )


# speedup vs baseline: 1.7308x; 1.4347x over previous
import functools

import jax
import jax.numpy as jnp
from jax import lax
from jax.experimental import pallas as pl
from jax.experimental.pallas import tpu as pltpu
from jax.experimental.pallas import tpu_sc as plsc

D = 32
BB = 128
L = 16
TP = BB + 1


def _build(B, H):
    N = B * H
    info = plsc.get_sparse_core_info()
    NC, NS = info.num_cores, info.num_subcores
    NW = NC * NS
    n_units = N // BB
    u_per_w = n_units // NW
    assert n_units % NW == 0 and B % BB == 0
    idx_per_w = N // NW
    NBUF = 2
    n_steps = u_per_w // NBUF
    BT = B // BB
    CT = D // 8

    mesh = plsc.VectorSubcoreMesh(core_axis_name="c", subcore_axis_name="s")

    @functools.partial(
        pl.kernel,
        mesh=mesh,
        out_type=jax.ShapeDtypeStruct((H, CT, BT, 8, BB), jnp.float32),
        compiler_params=pltpu.CompilerParams(
            use_tc_tiling_on_sc=False, needs_layout_passes=False
        ),
        scratch_types=[
            pltpu.VMEM((idx_per_w,), jnp.int32),
            pltpu.VMEM((NBUF, BB, D), jnp.float32),
            pltpu.VMEM((NBUF, CT, 8, TP), jnp.float32),
            pltpu.SemaphoreType.DMA,
            pltpu.SemaphoreType.DMA,
            pltpu.SemaphoreType.DMA,
            pltpu.SemaphoreType.DMA,
        ],
    )
    def k(idx_hbm, table_hbm, out_hbm, idx_v, rows_v, tr_v, g0, g1, o0, o1):
        gsem = (g0, g1)
        osem = (o0, o1)
        wid = lax.axis_index("s") * NC + lax.axis_index("c")
        base = wid * idx_per_w
        u0 = wid * u_per_w
        pltpu.sync_copy(idx_hbm.at[pl.ds(base, idx_per_w)], idx_v)

        lane = lax.iota(jnp.int32, L)
        ct_vec = [lane // 8, lane // 8 + 2]
        cs_vec = lane % 8

        def gather_copy(ul, b):
            off = pl.multiple_of(ul * BB, BB)
            return pltpu.make_async_copy(
                table_hbm.at[idx_v.at[pl.ds(off, BB)]], rows_v.at[b], gsem[b]
            )

        def store_copy(ul, b):
            u = u0 + ul
            h = lax.div(u, BT)
            bt = lax.rem(u, BT)
            return pltpu.make_async_copy(
                tr_v.at[b, :, :, pl.ds(0, BB)], out_hbm.at[h, :, bt], osem[b]
            )

        def transpose(b):
            dst = tr_v.at[b]
            for j in range(BB):
                jv = jnp.full((L,), j, jnp.int32)
                v0 = rows_v[b, j, pl.ds(0, L)]
                v1 = rows_v[b, j, pl.ds(L, L)]
                plsc.store_scatter(dst, [ct_vec[0], cs_vec, jv], v0)
                plsc.store_scatter(dst, [ct_vec[1], cs_vec, jv], v1)

        for b in range(NBUF):
            gather_copy(b, b).start()

        for b in range(NBUF):
            gather_copy(b, b).wait()
            transpose(b)
            gather_copy(b + NBUF, b).start()
            store_copy(b, b).start()

        def body(s, carry):
            for b in range(NBUF):
                ul = s * NBUF + b
                gather_copy(ul, b).wait()
                store_copy(ul - NBUF, b).wait()
                transpose(b)
                gather_copy(ul + NBUF, b).start()
                store_copy(ul, b).start()
            return carry

        lax.fori_loop(1, n_steps - 1, body, 0)

        for b in range(NBUF):
            ul = (n_steps - 1) * NBUF + b
            gather_copy(ul, b).wait()
            store_copy(ul - NBUF, b).wait()
            transpose(b)
            store_copy(ul, b).start()
        for b in range(NBUF):
            ul = (n_steps - 1) * NBUF + b
            store_copy(ul, b).wait()

    return k


def kernel(token_ids, emb):
    B, H = token_ids.shape
    idx = token_ids.T.reshape(B * H).astype(jnp.int32)
    k = _build(B, H)
    out5 = k(idx, emb)
    return out5.transpose(2, 4, 0, 1, 3).reshape(B, H, D)
